# baseline (device time: 68143 ns/iter reference)
import jax
import jax.numpy as jnp
from jax import lax
from jax.experimental import pallas as pl
from jax.experimental.pallas import tpu as pltpu

N_DEV = 16
B = 2
SQ = 128
SKV = 128
D = 512
HQ = 8
DH = 64
SCALE = 0.125

_RING = [0, 4, 8, 12, 15, 11, 7, 3, 2, 6, 10, 14, 13, 9, 5, 1]
_NEXT = [0] * N_DEV
_PREV = [0] * N_DEV
for _i, _l in enumerate(_RING):
    _NEXT[_l] = _RING[(_i + 1) % N_DEV]
    _PREV[_l] = _RING[(_i - 1) % N_DEV]

HOPS = 8

_STAGES = {
    2: [(6 * SKV, 5 * SKV)],
    4: [(4 * SKV, 2 * SKV), (11 * SKV, 2 * SKV)],
    6: [(2 * SKV, 2 * SKV), (13 * SKV, 2 * SKV)],
    8: [(0, 2 * SKV), (15 * SKV, SKV)],
}


def _lut(table, idx):
    out = jnp.int32(table[N_DEV - 1])
    for i in range(N_DEV - 1):
        out = jnp.where(idx == i, jnp.int32(table[i]), out)
    return out


def kernel(x, Wq, Wo, K_ext, V_ext):
    K2 = K_ext.reshape(B, SKV, HQ * DH)
    V2 = V_ext.reshape(B, SKV, HQ * DH)

    def body(x_ref, wq_ref, wo_ref, k_ref, v_ref, out_ref,
             kv_ref, r_send, r_recv, l_send, l_recv):
        my = lax.axis_index("i")
        nxt = _lut(_NEXT, my)
        prv = _lut(_PREV, my)

        rows = pl.ds(8 * SKV, SKV)
        kv_ref[rows, 0:512] = k_ref[0].astype(jnp.bfloat16)
        kv_ref[rows, 512:1024] = v_ref[0].astype(jnp.bfloat16)
        kv_ref[rows, 1024:1536] = k_ref[1].astype(jnp.bfloat16)
        kv_ref[rows, 1536:2048] = v_ref[1].astype(jnp.bfloat16)

        barrier_sem = pltpu.get_barrier_semaphore()
        for nbr in (prv, nxt):
            pl.semaphore_signal(
                barrier_sem, inc=1,
                device_id=(nbr,), device_id_type=pl.DeviceIdType.MESH,
            )
        pl.semaphore_wait(barrier_sem, 2)

        def mk_r(h, q):
            return pltpu.make_async_remote_copy(
                src_ref=kv_ref.at[pl.ds((9 - h) * SKV, SKV),
                                  pl.ds(q * 512, 512)],
                dst_ref=kv_ref.at[pl.ds((8 - h) * SKV, SKV),
                                  pl.ds(q * 512, 512)],
                send_sem=r_send.at[h - 1, q],
                recv_sem=r_recv.at[h - 1, q],
                device_id=(nxt,),
                device_id_type=pl.DeviceIdType.MESH,
            )

        def mk_l(h, q):
            dst_slot = 0 if h == 8 else 8 + h
            return pltpu.make_async_remote_copy(
                src_ref=kv_ref.at[pl.ds((7 + h) * SKV, SKV),
                                  pl.ds(q * 512, 512)],
                dst_ref=kv_ref.at[pl.ds(dst_slot * SKV, SKV),
                                  pl.ds(q * 512, 512)],
                send_sem=l_send.at[h - 1, q],
                recv_sem=l_recv.at[h - 1, q],
                device_id=(prv,),
                device_id_type=pl.DeviceIdType.MESH,
            )

        def cw_qs(h):
            return (0, 1) if h == HOPS else (0, 1, 2, 3)

        def ccw_qs(h):
            return (2, 3) if h == HOPS else (0, 1, 2, 3)

        r_rdmas = {(h, q): mk_r(h, q)
                   for h in range(1, HOPS + 1) for q in cw_qs(h)}
        l_rdmas = {(h, q): mk_l(h, q)
                   for h in range(1, HOPS + 1) for q in ccw_qs(h)}

        for q in range(4):
            r_rdmas[1, q].start()
            l_rdmas[1, q].start()

        qs = [
            (jnp.dot(x_ref[b], wq_ref[:, :],
                     preferred_element_type=jnp.float32) * SCALE
             ).astype(jnp.bfloat16)
            for b in range(B)
        ]

        state = {}

        def do_band(row0, nrows):
            for b in range(B):
                for hd in range(HQ):
                    q = qs[b][:, hd * DH:(hd + 1) * DH]
                    kcol = b * 1024 + hd * DH
                    k = kv_ref[row0:row0 + nrows, kcol:kcol + DH]
                    s = lax.dot_general(
                        q, k, (((1,), (1,)), ((), ())),
                        preferred_element_type=jnp.float32)
                    vcol = b * 1024 + 512 + hd * DH
                    v = kv_ref[row0:row0 + nrows, vcol:vcol + DH]
                    mj = jnp.max(s, axis=-1, keepdims=True)
                    if (b, hd) not in state:
                        p = jnp.exp(s - mj)
                        l = jnp.sum(p, axis=-1, keepdims=True)
                        acc = jnp.dot(p.astype(jnp.bfloat16), v,
                                      preferred_element_type=jnp.float32)
                        state[b, hd] = (mj, l, acc)
                    else:
                        m0, l0, a0 = state[b, hd]
                        mn = jnp.maximum(m0, mj)
                        alpha = jnp.exp(m0 - mn)
                        p = jnp.exp(s - mn)
                        l = l0 * alpha + jnp.sum(p, axis=-1, keepdims=True)
                        acc = a0 * alpha + jnp.dot(
                            p.astype(jnp.bfloat16), v,
                            preferred_element_type=jnp.float32)
                        state[b, hd] = (mn, l, acc)

        for h in range(1, HOPS + 1):
            for q in range(4):
                if (h, q) in r_rdmas:
                    r_rdmas[h, q].wait_recv()
                    if (h + 1, q) in r_rdmas:
                        r_rdmas[h + 1, q].start()
                if (h, q) in l_rdmas:
                    l_rdmas[h, q].wait_recv()
                    if (h + 1, q) in l_rdmas:
                        l_rdmas[h + 1, q].start()
            for row0, nrows in _STAGES.get(h, ()):
                do_band(row0, nrows)

        for b in range(B):
            heads = [state[b, hd][2] / state[b, hd][1] for hd in range(HQ)]
            attn = jnp.concatenate(heads, axis=1)
            out_ref[b] = jnp.dot(attn, wo_ref[:, :],
                                 preferred_element_type=jnp.float32)

        for rdma in r_rdmas.values():
            rdma.wait_send()
        for rdma in l_rdmas.values():
            rdma.wait_send()

    return pl.pallas_call(
        body,
        out_shape=jax.ShapeDtypeStruct((B, SQ, D), jnp.float32),
        in_specs=[pl.BlockSpec(memory_space=pltpu.VMEM)] * 5,
        out_specs=pl.BlockSpec(memory_space=pltpu.VMEM),
        scratch_shapes=[
            pltpu.VMEM((N_DEV * SKV, 2048), jnp.bfloat16),
            pltpu.SemaphoreType.DMA((HOPS, 4)),
            pltpu.SemaphoreType.DMA((HOPS, 4)),
            pltpu.SemaphoreType.DMA((HOPS, 4)),
            pltpu.SemaphoreType.DMA((HOPS, 4)),
        ],
        compiler_params=pltpu.CompilerParams(collective_id=0),
    )(x, Wq, Wo, K2, V2)


# device time: 65980 ns/iter; 1.0328x vs baseline; 1.0328x over previous
import jax
import jax.numpy as jnp
from jax import lax
from jax.experimental import pallas as pl
from jax.experimental.pallas import tpu as pltpu

N_DEV = 16
B = 2
SQ = 128
SKV = 128
D = 512
HQ = 8
DH = 64
SCALE = 0.125

_RING = [0, 4, 8, 12, 15, 11, 7, 3, 2, 6, 10, 14, 13, 9, 5, 1]
_NEXT = [0] * N_DEV
_PREV = [0] * N_DEV
for _i, _l in enumerate(_RING):
    _NEXT[_l] = _RING[(_i + 1) % N_DEV]
    _PREV[_l] = _RING[(_i - 1) % N_DEV]

HOPS = 8

_STAGES = {
    5: [(3 * SKV, 11 * SKV)],
    8: [(0, 3 * SKV), (14 * SKV, 2 * SKV)],
}


def _lut(table, idx):
    out = jnp.int32(table[N_DEV - 1])
    for i in range(N_DEV - 1):
        out = jnp.where(idx == i, jnp.int32(table[i]), out)
    return out


def kernel(x, Wq, Wo, K_ext, V_ext):
    K2 = K_ext.reshape(B, SKV, HQ * DH)
    V2 = V_ext.reshape(B, SKV, HQ * DH)

    def body(x_ref, wq_ref, wo_ref, k_ref, v_ref, out_ref,
             kv_ref, r_send, r_recv, l_send, l_recv):
        my = lax.axis_index("i")
        nxt = _lut(_NEXT, my)
        prv = _lut(_PREV, my)

        rows = pl.ds(8 * SKV, SKV)
        kv_ref[rows, 0:512] = k_ref[0].astype(jnp.bfloat16)
        kv_ref[rows, 512:1024] = v_ref[0].astype(jnp.bfloat16)
        kv_ref[rows, 1024:1536] = k_ref[1].astype(jnp.bfloat16)
        kv_ref[rows, 1536:2048] = v_ref[1].astype(jnp.bfloat16)

        barrier_sem = pltpu.get_barrier_semaphore()
        for nbr in (prv, nxt):
            pl.semaphore_signal(
                barrier_sem, inc=1,
                device_id=(nbr,), device_id_type=pl.DeviceIdType.MESH,
            )
        pl.semaphore_wait(barrier_sem, 2)

        def mk_r(h, q):
            return pltpu.make_async_remote_copy(
                src_ref=kv_ref.at[pl.ds((9 - h) * SKV, SKV),
                                  pl.ds(q * 512, 512)],
                dst_ref=kv_ref.at[pl.ds((8 - h) * SKV, SKV),
                                  pl.ds(q * 512, 512)],
                send_sem=r_send.at[h - 1, q],
                recv_sem=r_recv.at[h - 1, q],
                device_id=(nxt,),
                device_id_type=pl.DeviceIdType.MESH,
            )

        def mk_l(h, q):
            dst_slot = 0 if h == 8 else 8 + h
            return pltpu.make_async_remote_copy(
                src_ref=kv_ref.at[pl.ds((7 + h) * SKV, SKV),
                                  pl.ds(q * 512, 512)],
                dst_ref=kv_ref.at[pl.ds(dst_slot * SKV, SKV),
                                  pl.ds(q * 512, 512)],
                send_sem=l_send.at[h - 1, q],
                recv_sem=l_recv.at[h - 1, q],
                device_id=(prv,),
                device_id_type=pl.DeviceIdType.MESH,
            )

        def cw_qs(h):
            return (0, 1) if h == HOPS else (0, 1, 2, 3)

        def ccw_qs(h):
            return (2, 3) if h == HOPS else (0, 1, 2, 3)

        r_rdmas = {(h, q): mk_r(h, q)
                   for h in range(1, HOPS + 1) for q in cw_qs(h)}
        l_rdmas = {(h, q): mk_l(h, q)
                   for h in range(1, HOPS + 1) for q in ccw_qs(h)}

        for q in range(4):
            r_rdmas[1, q].start()
            l_rdmas[1, q].start()

        qs = [
            (jnp.dot(x_ref[b], wq_ref[:, :],
                     preferred_element_type=jnp.float32) * SCALE
             ).astype(jnp.bfloat16)
            for b in range(B)
        ]

        state = {}

        def do_band(row0, nrows):
            for b in range(B):
                for hd in range(HQ):
                    q = qs[b][:, hd * DH:(hd + 1) * DH]
                    kcol = b * 1024 + hd * DH
                    k = kv_ref[row0:row0 + nrows, kcol:kcol + DH]
                    s = lax.dot_general(
                        q, k, (((1,), (1,)), ((), ())),
                        preferred_element_type=jnp.float32)
                    vcol = b * 1024 + 512 + hd * DH
                    v = kv_ref[row0:row0 + nrows, vcol:vcol + DH]
                    mj = jnp.max(s, axis=-1, keepdims=True)
                    if (b, hd) not in state:
                        p = jnp.exp(s - mj)
                        l = jnp.sum(p, axis=-1, keepdims=True)
                        acc = jnp.dot(p.astype(jnp.bfloat16), v,
                                      preferred_element_type=jnp.float32)
                        state[b, hd] = (mj, l, acc)
                    else:
                        m0, l0, a0 = state[b, hd]
                        mn = jnp.maximum(m0, mj)
                        alpha = jnp.exp(m0 - mn)
                        p = jnp.exp(s - mn)
                        l = l0 * alpha + jnp.sum(p, axis=-1, keepdims=True)
                        acc = a0 * alpha + jnp.dot(
                            p.astype(jnp.bfloat16), v,
                            preferred_element_type=jnp.float32)
                        state[b, hd] = (mn, l, acc)

        for h in range(1, HOPS + 1):
            for q in range(4):
                if (h, q) in r_rdmas:
                    r_rdmas[h, q].wait_recv()
                    if (h + 1, q) in r_rdmas:
                        r_rdmas[h + 1, q].start()
                if (h, q) in l_rdmas:
                    l_rdmas[h, q].wait_recv()
                    if (h + 1, q) in l_rdmas:
                        l_rdmas[h + 1, q].start()
            for row0, nrows in _STAGES.get(h, ()):
                do_band(row0, nrows)

        for b in range(B):
            heads = [state[b, hd][2] / state[b, hd][1] for hd in range(HQ)]
            attn = jnp.concatenate(heads, axis=1)
            out_ref[b] = jnp.dot(attn, wo_ref[:, :],
                                 preferred_element_type=jnp.float32)

        for rdma in r_rdmas.values():
            rdma.wait_send()
        for rdma in l_rdmas.values():
            rdma.wait_send()

    return pl.pallas_call(
        body,
        out_shape=jax.ShapeDtypeStruct((B, SQ, D), jnp.float32),
        in_specs=[pl.BlockSpec(memory_space=pltpu.VMEM)] * 5,
        out_specs=pl.BlockSpec(memory_space=pltpu.VMEM),
        scratch_shapes=[
            pltpu.VMEM((N_DEV * SKV, 2048), jnp.bfloat16),
            pltpu.SemaphoreType.DMA((HOPS, 4)),
            pltpu.SemaphoreType.DMA((HOPS, 4)),
            pltpu.SemaphoreType.DMA((HOPS, 4)),
            pltpu.SemaphoreType.DMA((HOPS, 4)),
        ],
        compiler_params=pltpu.CompilerParams(collective_id=0),
    )(x, Wq, Wo, K2, V2)


# device time: 63034 ns/iter; 1.0811x vs baseline; 1.0467x over previous
import jax
import jax.numpy as jnp
from jax import lax
from jax.experimental import pallas as pl
from jax.experimental.pallas import tpu as pltpu

N_DEV = 16
B = 2
SQ = 128
SKV = 128
D = 512
HQ = 8
DH = 64
SCALE = 0.125

_RING = [0, 4, 8, 12, 15, 11, 7, 3, 2, 6, 10, 14, 13, 9, 5, 1]
_NEXT = [0] * N_DEV
_PREV = [0] * N_DEV
for _i, _l in enumerate(_RING):
    _NEXT[_l] = _RING[(_i + 1) % N_DEV]
    _PREV[_l] = _RING[(_i - 1) % N_DEV]

HOPS = 8

_STAGES = {
    8: [(0, N_DEV * SKV)],
}


def _lut(table, idx):
    out = jnp.int32(table[N_DEV - 1])
    for i in range(N_DEV - 1):
        out = jnp.where(idx == i, jnp.int32(table[i]), out)
    return out


def kernel(x, Wq, Wo, K_ext, V_ext):
    K2 = K_ext.reshape(B, SKV, HQ * DH)
    V2 = V_ext.reshape(B, SKV, HQ * DH)

    def body(x_ref, wq_ref, wo_ref, k_ref, v_ref, out_ref,
             kv_ref, r_send, r_recv, l_send, l_recv):
        my = lax.axis_index("i")
        nxt = _lut(_NEXT, my)
        prv = _lut(_PREV, my)

        rows = pl.ds(8 * SKV, SKV)
        kv_ref[rows, 0:512] = k_ref[0].astype(jnp.bfloat16)
        kv_ref[rows, 512:1024] = v_ref[0].astype(jnp.bfloat16)
        kv_ref[rows, 1024:1536] = k_ref[1].astype(jnp.bfloat16)
        kv_ref[rows, 1536:2048] = v_ref[1].astype(jnp.bfloat16)

        barrier_sem = pltpu.get_barrier_semaphore()
        for nbr in (prv, nxt):
            pl.semaphore_signal(
                barrier_sem, inc=1,
                device_id=(nbr,), device_id_type=pl.DeviceIdType.MESH,
            )
        pl.semaphore_wait(barrier_sem, 2)

        def mk_r(h, q):
            return pltpu.make_async_remote_copy(
                src_ref=kv_ref.at[pl.ds((9 - h) * SKV, SKV),
                                  pl.ds(q * 512, 512)],
                dst_ref=kv_ref.at[pl.ds((8 - h) * SKV, SKV),
                                  pl.ds(q * 512, 512)],
                send_sem=r_send.at[h - 1, q],
                recv_sem=r_recv.at[h - 1, q],
                device_id=(nxt,),
                device_id_type=pl.DeviceIdType.MESH,
            )

        def mk_l(h, q):
            dst_slot = 0 if h == 8 else 8 + h
            return pltpu.make_async_remote_copy(
                src_ref=kv_ref.at[pl.ds((7 + h) * SKV, SKV),
                                  pl.ds(q * 512, 512)],
                dst_ref=kv_ref.at[pl.ds(dst_slot * SKV, SKV),
                                  pl.ds(q * 512, 512)],
                send_sem=l_send.at[h - 1, q],
                recv_sem=l_recv.at[h - 1, q],
                device_id=(prv,),
                device_id_type=pl.DeviceIdType.MESH,
            )

        def cw_qs(h):
            return (0, 1) if h == HOPS else (0, 1, 2, 3)

        def ccw_qs(h):
            return (2, 3) if h == HOPS else (0, 1, 2, 3)

        r_rdmas = {(h, q): mk_r(h, q)
                   for h in range(1, HOPS + 1) for q in cw_qs(h)}
        l_rdmas = {(h, q): mk_l(h, q)
                   for h in range(1, HOPS + 1) for q in ccw_qs(h)}

        for q in range(4):
            r_rdmas[1, q].start()
            l_rdmas[1, q].start()

        wqb = wq_ref[:, :].astype(jnp.bfloat16)
        qs = [
            (jnp.dot(x_ref[b].astype(jnp.bfloat16), wqb,
                     preferred_element_type=jnp.float32) * SCALE
             ).astype(jnp.bfloat16)
            for b in range(B)
        ]

        state = {}

        def do_band(row0, nrows):
            for b in range(B):
                for hd in range(HQ):
                    q = qs[b][:, hd * DH:(hd + 1) * DH]
                    kcol = b * 1024 + hd * DH
                    k = kv_ref[row0:row0 + nrows, kcol:kcol + DH]
                    s = lax.dot_general(
                        q, k, (((1,), (1,)), ((), ())),
                        preferred_element_type=jnp.float32)
                    vcol = b * 1024 + 512 + hd * DH
                    v = kv_ref[row0:row0 + nrows, vcol:vcol + DH]
                    mj = jnp.max(s, axis=-1, keepdims=True)
                    if (b, hd) not in state:
                        p = jnp.exp(s - mj)
                        l = jnp.sum(p, axis=-1, keepdims=True)
                        acc = jnp.dot(p.astype(jnp.bfloat16), v,
                                      preferred_element_type=jnp.float32)
                        state[b, hd] = (mj, l, acc)
                    else:
                        m0, l0, a0 = state[b, hd]
                        mn = jnp.maximum(m0, mj)
                        alpha = jnp.exp(m0 - mn)
                        p = jnp.exp(s - mn)
                        l = l0 * alpha + jnp.sum(p, axis=-1, keepdims=True)
                        acc = a0 * alpha + jnp.dot(
                            p.astype(jnp.bfloat16), v,
                            preferred_element_type=jnp.float32)
                        state[b, hd] = (mn, l, acc)

        for h in range(1, HOPS + 1):
            for q in range(4):
                if (h, q) in r_rdmas:
                    r_rdmas[h, q].wait_recv()
                    if (h + 1, q) in r_rdmas:
                        r_rdmas[h + 1, q].start()
                if (h, q) in l_rdmas:
                    l_rdmas[h, q].wait_recv()
                    if (h + 1, q) in l_rdmas:
                        l_rdmas[h + 1, q].start()
            for row0, nrows in _STAGES.get(h, ()):
                do_band(row0, nrows)

        wob = wo_ref[:, :].astype(jnp.bfloat16)
        for b in range(B):
            heads = [state[b, hd][2] / state[b, hd][1] for hd in range(HQ)]
            attn = jnp.concatenate(heads, axis=1)
            out_ref[b] = jnp.dot(attn.astype(jnp.bfloat16), wob,
                                 preferred_element_type=jnp.float32)

        for rdma in r_rdmas.values():
            rdma.wait_send()
        for rdma in l_rdmas.values():
            rdma.wait_send()

    return pl.pallas_call(
        body,
        out_shape=jax.ShapeDtypeStruct((B, SQ, D), jnp.float32),
        in_specs=[pl.BlockSpec(memory_space=pltpu.VMEM)] * 5,
        out_specs=pl.BlockSpec(memory_space=pltpu.VMEM),
        scratch_shapes=[
            pltpu.VMEM((N_DEV * SKV, 2048), jnp.bfloat16),
            pltpu.SemaphoreType.DMA((HOPS, 4)),
            pltpu.SemaphoreType.DMA((HOPS, 4)),
            pltpu.SemaphoreType.DMA((HOPS, 4)),
            pltpu.SemaphoreType.DMA((HOPS, 4)),
        ],
        compiler_params=pltpu.CompilerParams(collective_id=0),
    )(x, Wq, Wo, K2, V2)
